# baseline (speedup 1.0000x reference)
import functools

import jax
import jax.numpy as jnp
from jax import lax
from jax.experimental import pallas as pl
from jax.experimental.pallas import tpu as pltpu
from jax.experimental.pallas import tpu_sc as plsc

_LN10 = 2.302585092994046

_ROWS = 128
_COLS = 128
_NW = 32
_RPW = _ROWS // _NW
_EPW = _RPW * _COLS
_LANES = 16

_MU = 0
_LS = _EPW
_NZ = 2 * _EPW
_OUT = 3 * _EPW


def _build():
    mesh = plsc.VectorSubcoreMesh(core_axis_name="c", subcore_axis_name="s")

    @functools.partial(
        pl.kernel,
        mesh=mesh,
        out_type=jax.ShapeDtypeStruct((_ROWS * _COLS,), jnp.float32),
        scratch_types=[
            pltpu.VMEM((_RPW, _COLS), jnp.int32),
            pltpu.VMEM((4 * _EPW,), jnp.float32),
            pltpu.SemaphoreType.DMA,
        ],
    )
    def k(ids_hbm, mu_hbm, ls_hbm, nz_hbm, out_hbm, idx_v, buf, sem):
        wid = lax.axis_index("s") * 2 + lax.axis_index("c")
        rbase = wid * _RPW
        ebase = wid * _EPW
        pltpu.sync_copy(ids_hbm.at[pl.ds(rbase, _RPW)], idx_v)
        for r in range(_RPW):
            pltpu.async_copy(
                mu_hbm.at[idx_v.at[r]], buf.at[pl.ds(_MU + r * _COLS, _COLS)], sem)
            pltpu.async_copy(
                ls_hbm.at[idx_v.at[r]], buf.at[pl.ds(_LS + r * _COLS, _COLS)], sem)
        pltpu.async_copy(nz_hbm.at[pl.ds(ebase, _EPW)], buf.at[pl.ds(_NZ, _EPW)], sem)
        pltpu.make_async_copy(
            mu_hbm.at[pl.ds(0, 3 * _EPW)], buf.at[pl.ds(0, 3 * _EPW)], sem
        ).wait()
        for i in range(_EPW // _LANES):
            sl = pl.ds(i * _LANES, _LANES)
            m = buf[pl.ds(_MU + i * _LANES, _LANES)]
            s = buf[pl.ds(_LS + i * _LANES, _LANES)]
            z = buf[pl.ds(_NZ + i * _LANES, _LANES)]
            buf[pl.ds(_OUT + i * _LANES, _LANES)] = jnp.exp(
                (m + jnp.exp(s) * z) * _LN10)
        pltpu.sync_copy(buf.at[pl.ds(_OUT, _EPW)], out_hbm.at[pl.ds(ebase, _EPW)])

    return k


_sc_kernel = _build()


def kernel(batch_size, family_ids, mu, log_sigma, noise):
    ids2 = family_ids.astype(jnp.int32).reshape(_ROWS, _COLS)
    out = _sc_kernel(ids2, mu, log_sigma, noise)
    return out


# speedup vs baseline: 1.0131x; 1.0131x over previous
import functools

import jax
import jax.numpy as jnp
from jax import lax
from jax.experimental import pallas as pl
from jax.experimental.pallas import tpu as pltpu
from jax.experimental.pallas import tpu_sc as plsc

_LN10 = 2.302585092994046

_ROWS = 128
_COLS = 128
_NW = 32
_RPW = _ROWS // _NW
_EPW = _RPW * _COLS
_LANES = 16


def _build():
    mesh = plsc.VectorSubcoreMesh(core_axis_name="c", subcore_axis_name="s")

    @functools.partial(
        pl.kernel,
        mesh=mesh,
        out_type=jax.ShapeDtypeStruct((_ROWS * _COLS,), jnp.float32),
        scratch_types=[
            pltpu.VMEM((_RPW, _COLS), jnp.int32),
            pltpu.VMEM((_EPW,), jnp.float32),
            pltpu.VMEM((_EPW,), jnp.float32),
            pltpu.VMEM((_EPW,), jnp.float32),
            pltpu.VMEM((_EPW,), jnp.float32),
            pltpu.SemaphoreType.DMA,
            pltpu.SemaphoreType.DMA,
        ],
    )
    def k(ids_hbm, mu_hbm, ls_hbm, nz_hbm, out_hbm,
          idx_v, mu_v, ls_v, nz_v, out_v, gsem, isem):
        wid = lax.axis_index("s") * 2 + lax.axis_index("c")
        rbase = wid * _RPW
        ebase = wid * _EPW
        nz_copy = pltpu.async_copy(nz_hbm.at[pl.ds(ebase, _EPW)], nz_v, isem)
        pltpu.sync_copy(ids_hbm.at[pl.ds(rbase, _RPW)], idx_v)
        gathers = []
        for r in range(_RPW):
            gathers.append(pltpu.async_copy(
                mu_hbm.at[idx_v.at[r]], mu_v.at[pl.ds(r * _COLS, _COLS)], gsem))
            gathers.append(pltpu.async_copy(
                ls_hbm.at[idx_v.at[r]], ls_v.at[pl.ds(r * _COLS, _COLS)], gsem))
        nz_copy.wait()
        for c in gathers:
            c.wait()

        def body(i, _):
            sl = pl.ds(pl.multiple_of(i * _LANES, _LANES), _LANES)
            out_v[sl] = jnp.exp((mu_v[sl] + jnp.exp(ls_v[sl]) * nz_v[sl]) * _LN10)
            return _

        lax.fori_loop(0, _EPW // _LANES, body, 0, unroll=4)
        pltpu.sync_copy(out_v, out_hbm.at[pl.ds(ebase, _EPW)])

    return k


_sc_kernel = _build()


def kernel(batch_size, family_ids, mu, log_sigma, noise):
    ids2 = family_ids.astype(jnp.int32).reshape(_ROWS, _COLS)
    out = _sc_kernel(ids2, mu, log_sigma, noise)
    return out
